# baseline (speedup 1.0000x reference)
import functools

import jax
import jax.numpy as jnp
from jax.experimental import pallas as pl
from jax.experimental.pallas import tpu as pltpu

B = 16
N = 16384
D = 256
H = 512
TILE = 1024


def _qnet_kernel(ps_ref, gemb_ref, w1b_ref, embed_ref, w1a_ref, w2_ref,
                 b2_ref, out_ref, g_scratch):
    i = pl.program_id(0)

    @pl.when(i == 0)
    def _():
        g_scratch[...] = jnp.dot(gemb_ref[...], w1b_ref[...],
                                 preferred_element_type=jnp.float32
                                 ).astype(jnp.bfloat16)

    rows = i * TILE + jax.lax.broadcasted_iota(jnp.int32, (TILE, B), 0)
    starts = ps_ref[0:1, :]
    ends = ps_ref[1:2, :]
    onehot = (rows >= starts) & (rows < ends)
    acc = jnp.dot(embed_ref[...], w1a_ref[...],
                  preferred_element_type=jnp.float32)
    acc = acc + jnp.dot(onehot.astype(jnp.bfloat16), g_scratch[...],
                        preferred_element_type=jnp.float32)
    h = jnp.maximum(acc, 0.0).astype(jnp.bfloat16)
    out_ref[...] = jnp.dot(h, w2_ref[...],
                           preferred_element_type=jnp.float32) + b2_ref[0, 0]


@jax.jit
def kernel(embed, graph_embed, prefix_sum, W1, b1, W2, b2):
    w1a = W1[:D].astype(jnp.bfloat16)
    w1b = W1[D:]
    embed_bf = embed.astype(jnp.bfloat16)
    w2_bf = W2.astype(jnp.bfloat16)
    gemb_aug = jnp.concatenate(
        [graph_embed, jnp.ones((B, 1), jnp.float32)], axis=1)
    w1b_aug = jnp.concatenate([w1b, b1[None, :]], axis=0)
    ends = prefix_sum.reshape(1, B)
    starts = jnp.concatenate(
        [jnp.zeros((1, 1), jnp.int32), ends[:, :-1]], axis=1)
    ps2d = jnp.concatenate([starts, ends], axis=0)
    b2_2d = b2.reshape(1, 1)

    grid = (N // TILE,)
    out = pl.pallas_call(
        _qnet_kernel,
        grid=grid,
        in_specs=[
            pl.BlockSpec((2, B), lambda i: (0, 0)),
            pl.BlockSpec((B, D + 1), lambda i: (0, 0)),
            pl.BlockSpec((D + 1, H), lambda i: (0, 0)),
            pl.BlockSpec((TILE, D), lambda i: (i, 0)),
            pl.BlockSpec((D, H), lambda i: (0, 0)),
            pl.BlockSpec((H, 1), lambda i: (0, 0)),
            pl.BlockSpec((1, 1), lambda i: (0, 0)),
        ],
        out_specs=pl.BlockSpec((TILE, 1), lambda i: (i, 0)),
        out_shape=jax.ShapeDtypeStruct((N, 1), jnp.float32),
        scratch_shapes=[pltpu.VMEM((B, H), jnp.bfloat16)],
    )(ps2d, gemb_aug, w1b_aug, embed_bf, w1a, w2_bf, b2_2d)
    return out


# speedup vs baseline: 1.1988x; 1.1988x over previous
import functools

import jax
import jax.numpy as jnp
from jax.experimental import pallas as pl
from jax.experimental.pallas import tpu as pltpu

B = 16
N = 16384
D = 256
H = 512
TILE = 1024


def _qnet_kernel(ps_ref, gemb_ref, w1b_ref, embed_ref, w1a_ref, w2_ref,
                 b2_ref, out_ref, g_scratch):
    i = pl.program_id(0)

    @pl.when(i == 0)
    def _():
        g_scratch[...] = jnp.dot(gemb_ref[...], w1b_ref[...],
                                 preferred_element_type=jnp.float32
                                 ).astype(jnp.bfloat16)

    rows = i * TILE + jax.lax.broadcasted_iota(jnp.int32, (TILE, B), 0)
    starts = ps_ref[0:1, :]
    ends = ps_ref[1:2, :]
    onehot = (rows >= starts) & (rows < ends)
    acc = jnp.dot(embed_ref[...].astype(jnp.bfloat16), w1a_ref[...],
                  preferred_element_type=jnp.float32)
    acc = acc + jnp.dot(onehot.astype(jnp.bfloat16), g_scratch[...],
                        preferred_element_type=jnp.float32)
    h = jnp.maximum(acc, 0.0).astype(jnp.bfloat16)
    out_ref[...] = jnp.dot(h, w2_ref[...],
                           preferred_element_type=jnp.float32) + b2_ref[0, 0]


@jax.jit
def kernel(embed, graph_embed, prefix_sum, W1, b1, W2, b2):
    w1a = W1[:D].astype(jnp.bfloat16)
    w1b = W1[D:]
    w2_bf = W2.astype(jnp.bfloat16)
    gemb_aug = jnp.concatenate(
        [graph_embed, jnp.ones((B, 1), jnp.float32)], axis=1)
    w1b_aug = jnp.concatenate([w1b, b1[None, :]], axis=0)
    ends = prefix_sum.reshape(1, B)
    starts = jnp.concatenate(
        [jnp.zeros((1, 1), jnp.int32), ends[:, :-1]], axis=1)
    ps2d = jnp.concatenate([starts, ends], axis=0)
    b2_2d = b2.reshape(1, 1)

    grid = (N // TILE,)
    out = pl.pallas_call(
        _qnet_kernel,
        grid=grid,
        in_specs=[
            pl.BlockSpec((2, B), lambda i: (0, 0)),
            pl.BlockSpec((B, D + 1), lambda i: (0, 0)),
            pl.BlockSpec((D + 1, H), lambda i: (0, 0)),
            pl.BlockSpec((TILE, D), lambda i: (i, 0)),
            pl.BlockSpec((D, H), lambda i: (0, 0)),
            pl.BlockSpec((H, 1), lambda i: (0, 0)),
            pl.BlockSpec((1, 1), lambda i: (0, 0)),
        ],
        out_specs=pl.BlockSpec((TILE, 1), lambda i: (i, 0)),
        out_shape=jax.ShapeDtypeStruct((N, 1), jnp.float32),
        scratch_shapes=[pltpu.VMEM((B, H), jnp.bfloat16)],
    )(ps2d, gemb_aug, w1b_aug, embed, w1a, w2_bf, b2_2d)
    return out


# speedup vs baseline: 1.6431x; 1.3706x over previous
import jax
import jax.numpy as jnp
from jax.experimental import pallas as pl
from jax.experimental.pallas import tpu as pltpu

B = 16
N = 16384
D = 256
H = 512
TILE = 2048


def _qnet_kernel(ps_ref, gemb_ref, w1_ref, b1_ref, w2_ref, b2_ref,
                 embed_ref, out_ref, g_scratch, w1a_scratch):
    i = pl.program_id(0)

    @pl.when(i == 0)
    def _():
        w1a_scratch[...] = w1_ref[:D, :].astype(jnp.bfloat16)
        g = jnp.dot(gemb_ref[...].astype(jnp.bfloat16),
                    w1_ref[D:, :].astype(jnp.bfloat16),
                    preferred_element_type=jnp.float32)
        g_scratch[...] = (g + b1_ref[...]).astype(jnp.bfloat16)

    rows = i * TILE + jax.lax.broadcasted_iota(jnp.int32, (TILE, B), 0)
    ends = ps_ref[...]
    seg = jnp.sum((rows >= ends).astype(jnp.int32), axis=1, keepdims=True)
    onehot = seg == jax.lax.broadcasted_iota(jnp.int32, (TILE, B), 1)

    acc = jnp.dot(embed_ref[...].astype(jnp.bfloat16), w1a_scratch[...],
                  preferred_element_type=jnp.float32)
    acc = acc + jnp.dot(onehot.astype(jnp.bfloat16), g_scratch[...],
                        preferred_element_type=jnp.float32)
    h = jnp.maximum(acc, 0.0).astype(jnp.bfloat16)
    out_ref[...] = jnp.dot(h, w2_ref[...].astype(jnp.bfloat16),
                           preferred_element_type=jnp.float32) + b2_ref[0, 0]


@jax.jit
def kernel(embed, graph_embed, prefix_sum, W1, b1, W2, b2):
    grid = (N // TILE,)
    out = pl.pallas_call(
        _qnet_kernel,
        grid=grid,
        in_specs=[
            pl.BlockSpec((1, B), lambda i: (0, 0)),
            pl.BlockSpec((B, D), lambda i: (0, 0)),
            pl.BlockSpec((2 * D, H), lambda i: (0, 0)),
            pl.BlockSpec((1, H), lambda i: (0, 0)),
            pl.BlockSpec((H, 1), lambda i: (0, 0)),
            pl.BlockSpec((1, 1), lambda i: (0, 0)),
            pl.BlockSpec((TILE, D), lambda i: (i, 0)),
        ],
        out_specs=pl.BlockSpec((TILE, 1), lambda i: (i, 0)),
        out_shape=jax.ShapeDtypeStruct((N, 1), jnp.float32),
        scratch_shapes=[pltpu.VMEM((B, H), jnp.bfloat16),
                        pltpu.VMEM((D, H), jnp.bfloat16)],
    )(prefix_sum.reshape(1, B), graph_embed, W1, b1.reshape(1, H),
      W2, b2.reshape(1, 1), embed)
    return out
